# baseline (speedup 1.0000x reference)
import functools

import jax
import jax.numpy as jnp
from jax import lax
from jax.experimental import pallas as pl
from jax.experimental.pallas import tpu as pltpu
from jax.experimental.pallas import tpu_sc as plsc

D = 64
SCALE = 8.0
NC = 2
NS = 16
NW = NC * NS
R = 256
NBUF = 2
L = 16
ABLK = 1024
PSTRIDE = R + 1


def _wide_transpose(table_t, V):
    nblk = (V + ABLK - 1) // ABLK
    W = nblk * (ABLK // 2)

    def body(x_ref, o_ref):
        t = x_ref[...].T
        o_ref[...] = jnp.concatenate(
            [t[0:ABLK // 2, :], t[ABLK // 2:ABLK, :]], axis=1)

    return pl.pallas_call(
        body,
        grid=(nblk,),
        in_specs=[pl.BlockSpec((D, ABLK), lambda g: (0, g))],
        out_specs=pl.BlockSpec((ABLK // 2, 2 * D), lambda g: (g, 0)),
        out_shape=jax.ShapeDtypeStruct((W, 2 * D), jnp.float32),
    )(table_t)


def _make_sc_embed(B0, B1, W):
    n_iblk = B0 // R
    n_tasks = B1 * n_iblk
    tpw = n_tasks // NW
    assert n_tasks % NW == 0 and tpw % NBUF == 0

    mesh = plsc.VectorSubcoreMesh(core_axis_name="c", subcore_axis_name="s")

    @functools.partial(
        pl.kernel,
        out_type=jax.ShapeDtypeStruct((B1, D, B0), jnp.float32),
        mesh=mesh,
        scratch_types=[
            [pltpu.VMEM((R,), jnp.int32) for _ in range(NBUF)],
            [pltpu.VMEM((R,), jnp.int32) for _ in range(NBUF)],
            [pltpu.VMEM((R, 2 * D), jnp.float32) for _ in range(NBUF)],
            [pltpu.VMEM((D, PSTRIDE), jnp.float32) for _ in range(NBUF)],
            [pltpu.SemaphoreType.DMA for _ in range(NBUF)],
            [pltpu.SemaphoreType.DMA for _ in range(NBUF)],
            [pltpu.SemaphoreType.DMA for _ in range(NBUF)],
        ],
        compiler_params=pltpu.CompilerParams(
            use_tc_tiling_on_sc=False, needs_layout_passes=False),
    )
    def sc_embed(wide_hbm, idxt_hbm, p_hbm, idx_v, gidx_v, wide_v, p_v,
                 isems, gsems, osems):
        wid = lax.axis_index("s") * NC + lax.axis_index("c")
        t0 = wid * tpw
        iota = lax.iota(jnp.int32, L)
        zv = iota * 0
        rowc = tuple(c * L + iota for c in range(D // L))

        def idx_src(t):
            off = pl.multiple_of((t0 + t) * R, R)
            return idxt_hbm.at[pl.ds(off, R)]

        def start_idx(t, b):
            pltpu.async_copy(idx_src(t), idx_v[b], isems[b])

        def start_gathers(t, b):
            pltpu.make_async_copy(idx_src(t), idx_v[b], isems[b]).wait()
            for u in range(R // L):
                iv = idx_v[b][pl.ds(u * L, L)]
                gidx_v[b][pl.ds(u * L, L)] = lax.bitwise_or(
                    lax.shift_left(lax.shift_right_logical(iv, 10), 9),
                    lax.bitwise_and(iv, 511))
            for h in range(R // 128):
                pltpu.async_copy(
                    wide_hbm.at[gidx_v[b].at[pl.ds(h * 128, 128)]],
                    wide_v[b].at[pl.ds(h * 128, 128)], gsems[b])

        def wait_gathers(b):
            for h in range(R // 128):
                pltpu.make_async_copy(
                    wide_hbm.at[gidx_v[b].at[pl.ds(h * 128, 128)]],
                    wide_v[b].at[pl.ds(h * 128, 128)], gsems[b]).wait()

        def compute(b):
            @plsc.parallel_loop(0, R // L)
            def group(g):
                iv = idx_v[b][pl.ds(g * L, L)]
                cb = lax.mul(lax.bitwise_and(lax.shift_right_logical(iv, 9), 1), D)
                for j in range(L):
                    r = g * L + j
                    s = cb[j]
                    colv = zv + r
                    for c in range(D // L):
                        v = wide_v[b][r, pl.ds(s + c * L, L)]
                        plsc.store_scatter(p_v[b], [rowc[c], colv], v * SCALE)

        def p_dst(t):
            tg = t0 + t
            j = lax.div(tg, n_iblk)
            i0 = pl.multiple_of(lax.rem(tg, n_iblk) * R, R)
            return p_hbm.at[j, :, pl.ds(i0, R)]

        def start_out(t, b):
            pltpu.async_copy(p_v[b].at[:, pl.ds(0, R)], p_dst(t), osems[b])

        def wait_out(t, b):
            pltpu.make_async_copy(
                p_v[b].at[:, pl.ds(0, R)], p_dst(t), osems[b]).wait()

        start_idx(0, 0)
        start_gathers(0, 0)
        start_idx(1, 1)

        def body(step, carry):
            tb = step * NBUF
            for b in range(NBUF):
                t = tb + b
                wait_gathers(b)

                @pl.when(t + 1 < tpw)
                def _():
                    start_gathers(t + 1, 1 - b)

                @pl.when(t >= NBUF)
                def _():
                    wait_out(t - NBUF, b)
                compute(b)
                start_out(t, b)

                @pl.when(t + NBUF < tpw)
                def _():
                    start_idx(t + NBUF, b)
            return carry
        lax.fori_loop(0, tpw // NBUF, body, 0)

        for b in range(NBUF):
            wait_out(tpw - NBUF + b, b)

    return sc_embed


def kernel(x, table):
    B0, B1 = x.shape
    V, d = table.shape
    wide = _wide_transpose(table.T, V)
    idxt_flat = x.T.reshape(B0 * B1)
    p = _make_sc_embed(B0, B1, wide.shape[0])(wide, idxt_flat)
    return p.transpose(2, 0, 1)


# speedup vs baseline: 1.0235x; 1.0235x over previous
import functools

import jax
import jax.numpy as jnp
from jax import lax
from jax.experimental import pallas as pl
from jax.experimental.pallas import tpu as pltpu
from jax.experimental.pallas import tpu_sc as plsc

D = 64
SCALE = 8.0
NC = 2
NS = 16
NW = NC * NS
R = 256
NBUF = 2
L = 16
ABLK = 1024
PSTRIDE = R + 1


def _wide_transpose(table_t, V):
    nblk = (V + ABLK - 1) // ABLK
    W = nblk * (ABLK // 2)

    def body(x_ref, o_ref):
        t = x_ref[...].T
        o_ref[...] = jnp.concatenate(
            [t[0:ABLK // 2, :], t[ABLK // 2:ABLK, :]], axis=1)

    return pl.pallas_call(
        body,
        grid=(nblk,),
        in_specs=[pl.BlockSpec((D, ABLK), lambda g: (0, g))],
        out_specs=pl.BlockSpec((ABLK // 2, 2 * D), lambda g: (g, 0)),
        out_shape=jax.ShapeDtypeStruct((W, 2 * D), jnp.float32),
    )(table_t)


def _make_sc_embed(B0, B1, W):
    n_iblk = B0 // R
    n_tasks = B1 * n_iblk
    tpw = n_tasks // NW
    assert n_tasks % NW == 0 and tpw % NBUF == 0

    mesh = plsc.VectorSubcoreMesh(core_axis_name="c", subcore_axis_name="s")

    @functools.partial(
        pl.kernel,
        out_type=jax.ShapeDtypeStruct((B1, D, B0), jnp.float32),
        mesh=mesh,
        scratch_types=[
            [pltpu.VMEM((R,), jnp.int32) for _ in range(NBUF)],
            [pltpu.VMEM((R,), jnp.int32) for _ in range(NBUF)],
            [pltpu.VMEM((R, D), jnp.float32) for _ in range(NBUF)],
            [pltpu.VMEM((D, PSTRIDE), jnp.float32) for _ in range(NBUF)],
            [pltpu.SemaphoreType.DMA for _ in range(NBUF)],
            [pltpu.SemaphoreType.DMA for _ in range(NBUF)],
            [pltpu.SemaphoreType.DMA for _ in range(NBUF)],
        ],
        compiler_params=pltpu.CompilerParams(
            use_tc_tiling_on_sc=False, needs_layout_passes=False),
    )
    def sc_embed(wide_hbm, idxt_hbm, p_hbm, idx_v, gidx_v, wide_v, p_v,
                 isems, gsems, osems):
        wid = lax.axis_index("s") * NC + lax.axis_index("c")
        t0 = wid * tpw
        iota = lax.iota(jnp.int32, L)
        zv = iota * 0
        rowc = tuple(c * L + iota for c in range(D // L))

        def idx_src(t):
            off = pl.multiple_of((t0 + t) * R, R)
            return idxt_hbm.at[pl.ds(off, R)]

        def start_idx(t, b):
            pltpu.async_copy(idx_src(t), idx_v[b], isems[b])

        def start_gathers(t, b):
            pltpu.make_async_copy(idx_src(t), idx_v[b], isems[b]).wait()
            for u in range(R // L):
                iv = idx_v[b][pl.ds(u * L, L)]
                gidx_v[b][pl.ds(u * L, L)] = lax.bitwise_or(
                    lax.bitwise_or(
                        lax.shift_left(lax.shift_right_logical(iv, 10), 10),
                        lax.shift_left(lax.bitwise_and(iv, 511), 1)),
                    lax.bitwise_and(lax.shift_right_logical(iv, 9), 1))
            for h in range(R // 128):
                pltpu.async_copy(
                    wide_hbm.at[gidx_v[b].at[pl.ds(h * 128, 128)]],
                    wide_v[b].at[pl.ds(h * 128, 128)], gsems[b])

        def wait_gathers(b):
            for h in range(R // 128):
                pltpu.make_async_copy(
                    wide_hbm.at[gidx_v[b].at[pl.ds(h * 128, 128)]],
                    wide_v[b].at[pl.ds(h * 128, 128)], gsems[b]).wait()

        def compute(b):
            @plsc.parallel_loop(0, R // L)
            def group(g):
                for j in range(L):
                    r = g * L + j
                    colv = zv + r
                    for c in range(D // L):
                        v = wide_v[b][r, pl.ds(c * L, L)]
                        plsc.store_scatter(p_v[b], [rowc[c], colv], v * SCALE)

        def p_dst(t):
            tg = t0 + t
            j = lax.div(tg, n_iblk)
            i0 = pl.multiple_of(lax.rem(tg, n_iblk) * R, R)
            return p_hbm.at[j, :, pl.ds(i0, R)]

        def start_out(t, b):
            pltpu.async_copy(p_v[b].at[:, pl.ds(0, R)], p_dst(t), osems[b])

        def wait_out(t, b):
            pltpu.make_async_copy(
                p_v[b].at[:, pl.ds(0, R)], p_dst(t), osems[b]).wait()

        start_idx(0, 0)
        start_gathers(0, 0)
        start_idx(1, 1)

        def body(step, carry):
            tb = step * NBUF
            for b in range(NBUF):
                t = tb + b
                wait_gathers(b)

                @pl.when(t + 1 < tpw)
                def _():
                    start_gathers(t + 1, 1 - b)

                @pl.when(t >= NBUF)
                def _():
                    wait_out(t - NBUF, b)
                compute(b)
                start_out(t, b)

                @pl.when(t + NBUF < tpw)
                def _():
                    start_idx(t + NBUF, b)
            return carry
        lax.fori_loop(0, tpw // NBUF, body, 0)

        for b in range(NBUF):
            wait_out(tpw - NBUF + b, b)

    return sc_embed


def kernel(x, table):
    B0, B1 = x.shape
    V, d = table.shape
    wide = _wide_transpose(table.T, V)
    rows = wide.reshape(2 * wide.shape[0], d)
    idxt_flat = x.T.reshape(B0 * B1)
    p = _make_sc_embed(B0, B1, rows.shape[0])(rows, idxt_flat)
    return p.transpose(2, 0, 1)


# speedup vs baseline: 1.2189x; 1.1909x over previous
import functools

import jax
import jax.numpy as jnp
from jax import lax
from jax.experimental import pallas as pl
from jax.experimental.pallas import tpu as pltpu
from jax.experimental.pallas import tpu_sc as plsc

D = 64
SCALE = 8.0
NC = 2
NS = 16
NW = NC * NS
R = 256
NBUF = 2
L = 16
ABLK = 1024
PSTRIDE = R + 1


def _wide_transpose(table_t, V):
    nblk = (V + ABLK - 1) // ABLK
    W = nblk * (ABLK // 2)

    def body(x_ref, o_ref):
        t = x_ref[...].T
        o_ref[...] = jnp.concatenate(
            [t[0:ABLK // 2, :], t[ABLK // 2:ABLK, :]], axis=1)

    return pl.pallas_call(
        body,
        grid=(nblk,),
        in_specs=[pl.BlockSpec((D, ABLK), lambda g: (0, g))],
        out_specs=pl.BlockSpec((ABLK // 2, 2 * D), lambda g: (g, 0)),
        out_shape=jax.ShapeDtypeStruct((W, 2 * D), jnp.float32),
    )(table_t)


def _make_sc_embed(B0, B1, W):
    n_iblk = B0 // R
    n_tasks = B1 * n_iblk
    tpw = n_tasks // NW
    assert n_tasks % NW == 0 and tpw % NBUF == 0

    mesh = plsc.VectorSubcoreMesh(core_axis_name="c", subcore_axis_name="s")

    @functools.partial(
        pl.kernel,
        out_type=jax.ShapeDtypeStruct((B1, 8, B0 // 128, 8, 128), jnp.float32),
        mesh=mesh,
        scratch_types=[
            [pltpu.VMEM((R,), jnp.int32) for _ in range(NBUF)],
            [pltpu.VMEM((R,), jnp.int32) for _ in range(NBUF)],
            [pltpu.VMEM((R, D), jnp.float32) for _ in range(NBUF)],
            [pltpu.VMEM((8, 8, 273), jnp.float32) for _ in range(NBUF)],
            [pltpu.SemaphoreType.DMA for _ in range(NBUF)],
            [pltpu.SemaphoreType.DMA for _ in range(NBUF)],
            [pltpu.SemaphoreType.DMA for _ in range(NBUF)],
        ],
        compiler_params=pltpu.CompilerParams(
            use_tc_tiling_on_sc=False, needs_layout_passes=False),
    )
    def sc_embed(wide_hbm, idxt_hbm, p_hbm, idx_v, gidx_v, wide_v, p_v,
                 isems, gsems, osems):
        wid = lax.axis_index("s") * NC + lax.axis_index("c")
        t0 = wid * tpw
        iota = lax.iota(jnp.int32, L)
        zv = iota * 0
        rowc = tuple(c * L + iota for c in range(D // L))
        av = tuple(lax.shift_right_logical(v, 3) for v in rowc)
        rv = tuple(lax.bitwise_and(v, 7) for v in rowc)

        def idx_src(t):
            off = pl.multiple_of((t0 + t) * R, R)
            return idxt_hbm.at[pl.ds(off, R)]

        def start_idx(t, b):
            pltpu.async_copy(idx_src(t), idx_v[b], isems[b])

        def start_gathers(t, b):
            pltpu.make_async_copy(idx_src(t), idx_v[b], isems[b]).wait()
            for u in range(R // L):
                iv = idx_v[b][pl.ds(u * L, L)]
                gidx_v[b][pl.ds(u * L, L)] = lax.bitwise_or(
                    lax.bitwise_or(
                        lax.shift_left(lax.shift_right_logical(iv, 10), 10),
                        lax.shift_left(lax.bitwise_and(iv, 511), 1)),
                    lax.bitwise_and(lax.shift_right_logical(iv, 9), 1))
            for h in range(R // 128):
                pltpu.async_copy(
                    wide_hbm.at[gidx_v[b].at[pl.ds(h * 128, 128)]],
                    wide_v[b].at[pl.ds(h * 128, 128)], gsems[b])

        def wait_gathers(b):
            for h in range(R // 128):
                pltpu.make_async_copy(
                    wide_hbm.at[gidx_v[b].at[pl.ds(h * 128, 128)]],
                    wide_v[b].at[pl.ds(h * 128, 128)], gsems[b]).wait()

        def compute(b):
            @plsc.parallel_loop(0, R // L)
            def group(g):
                for j in range(L):
                    r = g * L + j
                    sc = lax.mul(lax.shift_right_logical(r, 7), 136) + \
                        lax.bitwise_and(r, 127)
                    ccv = zv + sc
                    for c in range(D // L):
                        v = wide_v[b][r, pl.ds(c * L, L)]
                        plsc.store_scatter(p_v[b], [av[c], rv[c], ccv], v * SCALE)

        def p_dst(t, bb):
            tg = t0 + t
            j = lax.div(tg, n_iblk)
            bbg = lax.rem(tg, n_iblk) * (R // 128) + bb
            return p_hbm.at[j, :, bbg, :, :]

        def start_out(t, b):
            for bb in range(R // 128):
                pltpu.async_copy(
                    p_v[b].at[:, :, pl.ds(bb * 136, 128)], p_dst(t, bb),
                    osems[b])

        def wait_out(t, b):
            for bb in range(R // 128):
                pltpu.make_async_copy(
                    p_v[b].at[:, :, pl.ds(bb * 136, 128)], p_dst(t, bb),
                    osems[b]).wait()

        start_idx(0, 0)
        start_gathers(0, 0)
        start_idx(1, 1)

        def body(step, carry):
            tb = step * NBUF
            for b in range(NBUF):
                t = tb + b
                wait_gathers(b)

                @pl.when(t + 1 < tpw)
                def _():
                    start_gathers(t + 1, 1 - b)

                @pl.when(t >= NBUF)
                def _():
                    wait_out(t - NBUF, b)
                compute(b)
                start_out(t, b)

                @pl.when(t + NBUF < tpw)
                def _():
                    start_idx(t + NBUF, b)
            return carry
        lax.fori_loop(0, tpw // NBUF, body, 0)

        for b in range(NBUF):
            wait_out(tpw - NBUF + b, b)

    return sc_embed


def kernel(x, table):
    B0, B1 = x.shape
    V, d = table.shape
    wide = _wide_transpose(table.T, V)
    rows = wide.reshape(2 * wide.shape[0], d)
    idxt_flat = x.T.reshape(B0 * B1)
    p5 = _make_sc_embed(B0, B1, rows.shape[0])(rows, idxt_flat)
    p3 = p5.transpose(0, 1, 3, 2, 4).reshape(B1, d, B0)
    return p3.transpose(2, 0, 1)


# speedup vs baseline: 1.2191x; 1.0001x over previous
import functools

import jax
import jax.numpy as jnp
from jax import lax
from jax.experimental import pallas as pl
from jax.experimental.pallas import tpu as pltpu
from jax.experimental.pallas import tpu_sc as plsc

D = 64
SCALE = 8.0
NC = 2
NS = 16
NW = NC * NS
R = 256
NBUF = 2
L = 16
ABLK = 1024
PSTRIDE = R + 1


def _wide_transpose(table_t, V):
    nblk = (V + ABLK - 1) // ABLK
    W = nblk * (ABLK // 2)

    def body(x_ref, o_ref):
        t = x_ref[...].T
        o_ref[:, 0:D] = t[0:ABLK // 2, :]
        o_ref[:, D:2 * D] = t[ABLK // 2:ABLK, :]

    return pl.pallas_call(
        body,
        grid=(nblk,),
        in_specs=[pl.BlockSpec((D, ABLK), lambda g: (0, g))],
        out_specs=pl.BlockSpec((ABLK // 2, 2 * D), lambda g: (g, 0)),
        out_shape=jax.ShapeDtypeStruct((W, 2 * D), jnp.float32),
    )(table_t)


def _make_sc_embed(B0, B1, W):
    n_iblk = B0 // R
    n_tasks = B1 * n_iblk
    tpw = n_tasks // NW
    assert n_tasks % NW == 0 and tpw % NBUF == 0

    mesh = plsc.VectorSubcoreMesh(core_axis_name="c", subcore_axis_name="s")

    @functools.partial(
        pl.kernel,
        out_type=jax.ShapeDtypeStruct((B1, 8, B0 // 128, 8, 128), jnp.float32),
        mesh=mesh,
        scratch_types=[
            [pltpu.VMEM((R,), jnp.int32) for _ in range(NBUF)],
            [pltpu.VMEM((R,), jnp.int32) for _ in range(NBUF)],
            [pltpu.VMEM((R, D), jnp.float32) for _ in range(NBUF)],
            [pltpu.VMEM((8, 8, 273), jnp.float32) for _ in range(NBUF)],
            [pltpu.SemaphoreType.DMA for _ in range(NBUF)],
            [pltpu.SemaphoreType.DMA for _ in range(NBUF)],
            [pltpu.SemaphoreType.DMA for _ in range(NBUF)],
        ],
        compiler_params=pltpu.CompilerParams(
            use_tc_tiling_on_sc=False, needs_layout_passes=False),
    )
    def sc_embed(wide_hbm, idxt_hbm, p_hbm, idx_v, gidx_v, wide_v, p_v,
                 isems, gsems, osems):
        wid = lax.axis_index("s") * NC + lax.axis_index("c")
        t0 = wid * tpw
        iota = lax.iota(jnp.int32, L)
        zv = iota * 0
        rowc = tuple(c * L + iota for c in range(D // L))
        av = tuple(lax.shift_right_logical(v, 3) for v in rowc)
        rv = tuple(lax.bitwise_and(v, 7) for v in rowc)

        def idx_src(t):
            off = pl.multiple_of((t0 + t) * R, R)
            return idxt_hbm.at[pl.ds(off, R)]

        def start_idx(t, b):
            pltpu.async_copy(idx_src(t), idx_v[b], isems[b])

        def start_gathers(t, b):
            pltpu.make_async_copy(idx_src(t), idx_v[b], isems[b]).wait()
            for u in range(R // L):
                iv = idx_v[b][pl.ds(u * L, L)]
                gidx_v[b][pl.ds(u * L, L)] = lax.bitwise_or(
                    lax.bitwise_or(
                        lax.shift_left(lax.shift_right_logical(iv, 10), 10),
                        lax.shift_left(lax.bitwise_and(iv, 511), 1)),
                    lax.bitwise_and(lax.shift_right_logical(iv, 9), 1))
            for h in range(R // 128):
                pltpu.async_copy(
                    wide_hbm.at[gidx_v[b].at[pl.ds(h * 128, 128)]],
                    wide_v[b].at[pl.ds(h * 128, 128)], gsems[b])

        def wait_gathers(b):
            for h in range(R // 128):
                pltpu.make_async_copy(
                    wide_hbm.at[gidx_v[b].at[pl.ds(h * 128, 128)]],
                    wide_v[b].at[pl.ds(h * 128, 128)], gsems[b]).wait()

        def compute(b):
            @plsc.parallel_loop(0, R // L)
            def group(g):
                for j in range(L):
                    r = g * L + j
                    sc = lax.mul(lax.shift_right_logical(r, 7), 136) + \
                        lax.bitwise_and(r, 127)
                    ccv = zv + sc
                    for c in range(D // L):
                        v = wide_v[b][r, pl.ds(c * L, L)]
                        plsc.store_scatter(p_v[b], [av[c], rv[c], ccv], v * SCALE)

        def p_dst(t, bb):
            tg = t0 + t
            j = lax.div(tg, n_iblk)
            bbg = lax.rem(tg, n_iblk) * (R // 128) + bb
            return p_hbm.at[j, :, bbg, :, :]

        def start_out(t, b):
            for bb in range(R // 128):
                pltpu.async_copy(
                    p_v[b].at[:, :, pl.ds(bb * 136, 128)], p_dst(t, bb),
                    osems[b])

        def wait_out(t, b):
            for bb in range(R // 128):
                pltpu.make_async_copy(
                    p_v[b].at[:, :, pl.ds(bb * 136, 128)], p_dst(t, bb),
                    osems[b]).wait()

        start_idx(0, 0)
        start_gathers(0, 0)
        start_idx(1, 1)

        def body(step, carry):
            tb = step * NBUF
            for b in range(NBUF):
                t = tb + b
                wait_gathers(b)

                @pl.when(t + 1 < tpw)
                def _():
                    start_gathers(t + 1, 1 - b)

                @pl.when(t >= NBUF)
                def _():
                    wait_out(t - NBUF, b)
                compute(b)
                start_out(t, b)

                @pl.when(t + NBUF < tpw)
                def _():
                    start_idx(t + NBUF, b)
            return carry
        lax.fori_loop(0, tpw // NBUF, body, 0)

        for b in range(NBUF):
            wait_out(tpw - NBUF + b, b)

    return sc_embed


def kernel(x, table):
    B0, B1 = x.shape
    V, d = table.shape
    wide = _wide_transpose(table.T, V)
    rows = wide.reshape(2 * wide.shape[0], d)
    idxt_flat = x.T.reshape(B0 * B1)
    p5 = _make_sc_embed(B0, B1, rows.shape[0])(rows, idxt_flat)
    p3 = p5.transpose(0, 1, 3, 2, 4).reshape(B1, d, B0)
    return p3.transpose(2, 0, 1)


# speedup vs baseline: 1.5984x; 1.3111x over previous
import functools

import jax
import jax.numpy as jnp
from jax import lax
from jax.experimental import pallas as pl
from jax.experimental.pallas import tpu as pltpu
from jax.experimental.pallas import tpu_sc as plsc

D = 64
SCALE = 8.0
NC = 2
NS = 16
NW = NC * NS
R = 256
NBUF = 2
L = 16
ABLK = 2048
PSTRIDE = R + 1


def _wide_transpose(table_t, V):
    nblk = (V + ABLK - 1) // ABLK
    W = nblk * (ABLK // 2)

    def body(x_ref, o_ref):
        t = x_ref[...].T
        o_ref[:, 0:D] = t[0:ABLK // 2, :]
        o_ref[:, D:2 * D] = t[ABLK // 2:ABLK, :]

    return pl.pallas_call(
        body,
        grid=(nblk,),
        in_specs=[pl.BlockSpec((D, ABLK), lambda g: (0, g))],
        out_specs=pl.BlockSpec((ABLK // 2, 2 * D), lambda g: (g, 0)),
        out_shape=jax.ShapeDtypeStruct((W, 2 * D), jnp.float32),
    )(table_t)


def _make_sc_embed(B0, B1, W):
    n_iblk = B0 // R
    n_tasks = B1 * n_iblk
    tpw = n_tasks // NW
    assert n_tasks % NW == 0 and tpw % NBUF == 0

    mesh = plsc.VectorSubcoreMesh(core_axis_name="c", subcore_axis_name="s")

    @functools.partial(
        pl.kernel,
        out_type=jax.ShapeDtypeStruct((B1, 8, B0 // 128, 8, 128), jnp.float32),
        mesh=mesh,
        scratch_types=[
            [pltpu.VMEM((R,), jnp.int32) for _ in range(NBUF)],
            [pltpu.VMEM((R,), jnp.int32) for _ in range(NBUF)],
            [pltpu.VMEM((R, D), jnp.float32) for _ in range(NBUF)],
            [pltpu.VMEM((8, 8, 273), jnp.float32) for _ in range(NBUF)],
            [pltpu.SemaphoreType.DMA for _ in range(NBUF)],
            [pltpu.SemaphoreType.DMA for _ in range(NBUF)],
            [pltpu.SemaphoreType.DMA for _ in range(NBUF)],
        ],
        compiler_params=pltpu.CompilerParams(
            use_tc_tiling_on_sc=False, needs_layout_passes=False),
    )
    def sc_embed(wide_hbm, idxt_hbm, p_hbm, idx_v, gidx_v, wide_v, p_v,
                 isems, gsems, osems):
        wid = lax.axis_index("s") * NC + lax.axis_index("c")
        t0 = wid * tpw
        iota = lax.iota(jnp.int32, L)
        zv = iota * 0
        rowc = tuple(c * L + iota for c in range(D // L))
        av = tuple(lax.shift_right_logical(v, 3) for v in rowc)
        rv = tuple(lax.bitwise_and(v, 7) for v in rowc)

        def idx_src(t):
            off = pl.multiple_of((t0 + t) * R, R)
            return idxt_hbm.at[pl.ds(off, R)]

        def start_idx(t, b):
            pltpu.async_copy(idx_src(t), idx_v[b], isems[b])

        def start_gathers(t, b):
            pltpu.make_async_copy(idx_src(t), idx_v[b], isems[b]).wait()
            n = ABLK.bit_length() - 1
            for u in range(R // L):
                iv = idx_v[b][pl.ds(u * L, L)]
                gidx_v[b][pl.ds(u * L, L)] = lax.bitwise_or(
                    lax.bitwise_or(
                        lax.shift_left(lax.shift_right_logical(iv, n), n),
                        lax.shift_left(lax.bitwise_and(iv, ABLK // 2 - 1), 1)),
                    lax.bitwise_and(lax.shift_right_logical(iv, n - 1), 1))
            for h in range(R // 128):
                pltpu.async_copy(
                    wide_hbm.at[gidx_v[b].at[pl.ds(h * 128, 128)]],
                    wide_v[b].at[pl.ds(h * 128, 128)], gsems[b])

        def wait_gathers(b):
            for h in range(R // 128):
                pltpu.make_async_copy(
                    wide_hbm.at[gidx_v[b].at[pl.ds(h * 128, 128)]],
                    wide_v[b].at[pl.ds(h * 128, 128)], gsems[b]).wait()

        def compute(b):
            @plsc.parallel_loop(0, R // L)
            def group(g):
                for j in range(L):
                    r = g * L + j
                    sc = lax.mul(lax.shift_right_logical(r, 7), 136) + \
                        lax.bitwise_and(r, 127)
                    ccv = zv + sc
                    for c in range(D // L):
                        v = wide_v[b][r, pl.ds(c * L, L)]
                        plsc.store_scatter(p_v[b], [av[c], rv[c], ccv], v * SCALE)

        def p_dst(t, bb):
            tg = t0 + t
            j = lax.div(tg, n_iblk)
            bbg = lax.rem(tg, n_iblk) * (R // 128) + bb
            return p_hbm.at[j, :, bbg, :, :]

        def start_out(t, b):
            for bb in range(R // 128):
                pltpu.async_copy(
                    p_v[b].at[:, :, pl.ds(bb * 136, 128)], p_dst(t, bb),
                    osems[b])

        def wait_out(t, b):
            for bb in range(R // 128):
                pltpu.make_async_copy(
                    p_v[b].at[:, :, pl.ds(bb * 136, 128)], p_dst(t, bb),
                    osems[b]).wait()

        start_idx(0, 0)
        start_gathers(0, 0)
        start_idx(1, 1)

        def body(step, carry):
            tb = step * NBUF
            for b in range(NBUF):
                t = tb + b
                wait_gathers(b)

                @pl.when(t + 1 < tpw)
                def _():
                    start_gathers(t + 1, 1 - b)

                @pl.when(t >= NBUF)
                def _():
                    wait_out(t - NBUF, b)
                compute(b)
                start_out(t, b)

                @pl.when(t + NBUF < tpw)
                def _():
                    start_idx(t + NBUF, b)
            return carry
        lax.fori_loop(0, tpw // NBUF, body, 0)

        for b in range(NBUF):
            wait_out(tpw - NBUF + b, b)

    return sc_embed


def kernel(x, table):
    B0, B1 = x.shape
    V, d = table.shape
    wide = _wide_transpose(table.T, V)
    rows = wide.reshape(2 * wide.shape[0], d)
    idxt_flat = x.T.reshape(B0 * B1)
    p5 = _make_sc_embed(B0, B1, rows.shape[0])(rows, idxt_flat)
    p3 = p5.transpose(0, 1, 3, 2, 4).reshape(B1, d, B0)
    return p3.transpose(2, 0, 1)


# speedup vs baseline: 1.8879x; 1.1811x over previous
import functools

import jax
import jax.numpy as jnp
from jax import lax
from jax.experimental import pallas as pl
from jax.experimental.pallas import tpu as pltpu
from jax.experimental.pallas import tpu_sc as plsc

D = 64
SCALE = 8.0
NC = 2
NS = 16
NW = NC * NS
R = 256
NBUF = 2
L = 16
ABLK = 4096
PSTRIDE = R + 1


def _wide_transpose(table_t, V):
    nblk = (V + ABLK - 1) // ABLK
    W = nblk * (ABLK // 2)

    def body(x_ref, o_ref):
        t = x_ref[...].T
        o_ref[:, 0:D] = t[0:ABLK // 2, :]
        o_ref[:, D:2 * D] = t[ABLK // 2:ABLK, :]

    return pl.pallas_call(
        body,
        grid=(nblk,),
        in_specs=[pl.BlockSpec((D, ABLK), lambda g: (0, g))],
        out_specs=pl.BlockSpec((ABLK // 2, 2 * D), lambda g: (g, 0)),
        out_shape=jax.ShapeDtypeStruct((W, 2 * D), jnp.float32),
    )(table_t)


def _make_sc_embed(B0, B1, W):
    n_iblk = B0 // R
    n_tasks = B1 * n_iblk
    tpw = n_tasks // NW
    assert n_tasks % NW == 0 and tpw % NBUF == 0

    mesh = plsc.VectorSubcoreMesh(core_axis_name="c", subcore_axis_name="s")

    @functools.partial(
        pl.kernel,
        out_type=jax.ShapeDtypeStruct((B1, 8, B0 // 128, 8, 128), jnp.float32),
        mesh=mesh,
        scratch_types=[
            [pltpu.VMEM((R,), jnp.int32) for _ in range(NBUF)],
            [pltpu.VMEM((R,), jnp.int32) for _ in range(NBUF)],
            [pltpu.VMEM((R, D), jnp.float32) for _ in range(NBUF)],
            [pltpu.VMEM((8, 8, 273), jnp.float32) for _ in range(NBUF)],
            [pltpu.SemaphoreType.DMA for _ in range(NBUF)],
            [pltpu.SemaphoreType.DMA for _ in range(NBUF)],
            [pltpu.SemaphoreType.DMA for _ in range(NBUF)],
        ],
        compiler_params=pltpu.CompilerParams(
            use_tc_tiling_on_sc=False, needs_layout_passes=False),
    )
    def sc_embed(wide_hbm, idxt_hbm, p_hbm, idx_v, gidx_v, wide_v, p_v,
                 isems, gsems, osems):
        wid = lax.axis_index("s") * NC + lax.axis_index("c")
        t0 = wid * tpw
        iota = lax.iota(jnp.int32, L)
        zv = iota * 0
        rowc = tuple(c * L + iota for c in range(D // L))
        av = tuple(lax.shift_right_logical(v, 3) for v in rowc)
        rv = tuple(lax.bitwise_and(v, 7) for v in rowc)

        def idx_src(t):
            off = pl.multiple_of((t0 + t) * R, R)
            return idxt_hbm.at[pl.ds(off, R)]

        def start_idx(t, b):
            pltpu.async_copy(idx_src(t), idx_v[b], isems[b])

        def start_gathers(t, b):
            pltpu.make_async_copy(idx_src(t), idx_v[b], isems[b]).wait()
            n = ABLK.bit_length() - 1
            for u in range(R // L):
                iv = idx_v[b][pl.ds(u * L, L)]
                gidx_v[b][pl.ds(u * L, L)] = lax.bitwise_or(
                    lax.bitwise_or(
                        lax.shift_left(lax.shift_right_logical(iv, n), n),
                        lax.shift_left(lax.bitwise_and(iv, ABLK // 2 - 1), 1)),
                    lax.bitwise_and(lax.shift_right_logical(iv, n - 1), 1))
            for h in range(R // 128):
                pltpu.async_copy(
                    wide_hbm.at[gidx_v[b].at[pl.ds(h * 128, 128)]],
                    wide_v[b].at[pl.ds(h * 128, 128)], gsems[b])

        def wait_gathers(b):
            for h in range(R // 128):
                pltpu.make_async_copy(
                    wide_hbm.at[gidx_v[b].at[pl.ds(h * 128, 128)]],
                    wide_v[b].at[pl.ds(h * 128, 128)], gsems[b]).wait()

        def compute(b):
            @plsc.parallel_loop(0, R // L)
            def group(g):
                for j in range(L):
                    r = g * L + j
                    sc = lax.mul(lax.shift_right_logical(r, 7), 136) + \
                        lax.bitwise_and(r, 127)
                    ccv = zv + sc
                    for c in range(D // L):
                        v = wide_v[b][r, pl.ds(c * L, L)]
                        plsc.store_scatter(p_v[b], [av[c], rv[c], ccv], v * SCALE)

        def p_dst(t, bb):
            tg = t0 + t
            j = lax.div(tg, n_iblk)
            bbg = lax.rem(tg, n_iblk) * (R // 128) + bb
            return p_hbm.at[j, :, bbg, :, :]

        def start_out(t, b):
            for bb in range(R // 128):
                pltpu.async_copy(
                    p_v[b].at[:, :, pl.ds(bb * 136, 128)], p_dst(t, bb),
                    osems[b])

        def wait_out(t, b):
            for bb in range(R // 128):
                pltpu.make_async_copy(
                    p_v[b].at[:, :, pl.ds(bb * 136, 128)], p_dst(t, bb),
                    osems[b]).wait()

        start_idx(0, 0)
        start_gathers(0, 0)
        start_idx(1, 1)

        def body(step, carry):
            tb = step * NBUF
            for b in range(NBUF):
                t = tb + b
                wait_gathers(b)

                @pl.when(t + 1 < tpw)
                def _():
                    start_gathers(t + 1, 1 - b)

                @pl.when(t >= NBUF)
                def _():
                    wait_out(t - NBUF, b)
                compute(b)
                start_out(t, b)

                @pl.when(t + NBUF < tpw)
                def _():
                    start_idx(t + NBUF, b)
            return carry
        lax.fori_loop(0, tpw // NBUF, body, 0)

        for b in range(NBUF):
            wait_out(tpw - NBUF + b, b)

    return sc_embed


def kernel(x, table):
    B0, B1 = x.shape
    V, d = table.shape
    wide = _wide_transpose(table.T, V)
    rows = wide.reshape(2 * wide.shape[0], d)
    idxt_flat = x.T.reshape(B0 * B1)
    p5 = _make_sc_embed(B0, B1, rows.shape[0])(rows, idxt_flat)
    p3 = p5.transpose(0, 1, 3, 2, 4).reshape(B1, d, B0)
    return p3.transpose(2, 0, 1)
